# baseline (speedup 1.0000x reference)
import functools

import jax
import jax.numpy as jnp
from jax import lax
from jax.experimental import pallas as pl
from jax.experimental.pallas import tpu as pltpu
from jax.experimental.pallas import tpu_sc as plsc

_NC = 2
_NS = 16
_K = 128


def _sc_aggregate(x, src, dst, edge_attr):
    N, D = x.shape
    E = edge_attr.shape[0]
    K = _K
    W = _NC * _NS
    NCH = E // K
    JMAX = (NCH + W - 1) // W
    VEC = D // 16
    ZK = 80
    ZB = N // ZK
    ZPT = (ZB + _NS - 1) // _NS

    mesh = plsc.VectorSubcoreMesh(core_axis_name="c", subcore_axis_name="s",
                                  num_cores=_NC, num_subcores=_NS)

    @functools.partial(
        pl.kernel,
        out_type=jax.ShapeDtypeStruct((_NC, N, D), jnp.float32),
        mesh=mesh,
        scratch_types=[
            pltpu.VMEM((K,), jnp.int32),
            pltpu.VMEM((K,), jnp.int32),
            pltpu.VMEM((K, D), jnp.float32),
            pltpu.VMEM((K, D), jnp.float32),
            pltpu.VMEM((K, D), jnp.float32),
            pltpu.SemaphoreType.DMA,
            pltpu.SemaphoreType.DMA,
            pltpu.SemaphoreType.DMA,
            pltpu.SemaphoreType.DMA,
            pltpu.SemaphoreType.DMA,
            pltpu.SemaphoreType.DMA,
            pltpu.VMEM_SHARED((N, D), jnp.float32),
        ],
    )
    def agg(x_hbm, src_hbm, dst_hbm, ea_hbm, parts_hbm,
            src_k, dst_k, ea0, ea1, xg_v,
            si, sd, se0, se1, sg, ss, acc):
        c = lax.axis_index("c")
        s = lax.axis_index("s")
        w = c * _NS + s

        ea_v = (ea0, ea1)
        se = (se0, se1)

        zero = jnp.zeros((16,), jnp.float32)

        def zrow(r, carry):
            for t in range(VEC):
                xg_v[r, pl.ds(t * 16, 16)] = zero
            return carry

        lax.fori_loop(0, ZK, zrow, 0)
        for i in range(ZPT):
            blk = s * ZPT + i
            @pl.when(blk < ZB)
            def _():
                pltpu.sync_copy(xg_v.at[pl.ds(0, ZK)],
                                acc.at[pl.ds(blk * ZK, ZK)])
        plsc.subcore_barrier()

        @pl.when(w < NCH)
        def _():
            pltpu.async_copy(src_hbm.at[pl.ds(w * K, K)], src_k, si)
            pltpu.async_copy(dst_hbm.at[pl.ds(w * K, K)], dst_k, sd)
            pltpu.async_copy(ea_hbm.at[pl.ds(w * K, K)], ea0, se0)

        def chunk_body(j, carry):
            cid = j * W + w
            nbase = (cid + W) * K
            even = lax.rem(j, 2) == 0

            @pl.when((j > 0) & (cid - W < NCH))
            def _():
                pltpu.make_async_copy(xg_v, acc.at[dst_k], ss).wait()
                @pl.when(cid < NCH)
                def _():
                    pltpu.async_copy(dst_hbm.at[pl.ds(cid * K, K)],
                                     dst_k, sd)

            @pl.when(cid < NCH)
            def _():
                pltpu.make_async_copy(src_hbm.at[pl.ds(cid * K, K)],
                                      src_k, si).wait()
                pltpu.async_copy(x_hbm.at[src_k], xg_v, sg).wait()

                @pl.when(cid + W < NCH)
                def _():
                    pltpu.async_copy(src_hbm.at[pl.ds(nbase, K)], src_k, si)

                def relu_from(ea_ref):
                    @plsc.parallel_loop(0, K, 2, unroll=2)
                    def _(r):
                        for rr in range(2):
                            for t in range(VEC):
                                slc = pl.ds(t * 16, 16)
                                xg_v[r + rr, slc] = jnp.maximum(
                                    xg_v[r + rr, slc] + ea_ref[r + rr, slc],
                                    0.0)

                @pl.when(even)
                def _():
                    pltpu.make_async_copy(ea_hbm.at[pl.ds(cid * K, K)],
                                          ea0, se0).wait()
                    @pl.when(cid + W < NCH)
                    def _():
                        pltpu.async_copy(ea_hbm.at[pl.ds(nbase, K)], ea1, se1)
                    relu_from(ea0)

                @pl.when(jnp.logical_not(even))
                def _():
                    pltpu.make_async_copy(ea_hbm.at[pl.ds(cid * K, K)],
                                          ea1, se1).wait()
                    @pl.when(cid + W < NCH)
                    def _():
                        pltpu.async_copy(ea_hbm.at[pl.ds(nbase, K)], ea0, se0)
                    relu_from(ea1)

                pltpu.make_async_copy(dst_hbm.at[pl.ds(cid * K, K)],
                                      dst_k, sd).wait()
                pltpu.async_copy(xg_v, acc.at[dst_k], ss, add=True)

            return carry

        lax.fori_loop(0, JMAX + 1, chunk_body, 0)

        plsc.subcore_barrier()
        for i in range(ZPT):
            blk = s * ZPT + i
            @pl.when(blk < ZB)
            def _():
                pltpu.sync_copy(acc.at[pl.ds(blk * ZK, ZK)],
                                parts_hbm.at[c, pl.ds(blk * ZK, ZK)])

    return agg(x, src, dst, edge_attr)


def _mlp(x, parts, W1, b1, W2, b2, BN=1000):
    N, D = x.shape

    def body(x_ref, p_ref, w1_ref, b1_ref, w2_ref, b2_ref, o_ref):
        h = x_ref[...] + p_ref[0] + p_ref[1]
        h = lax.dot_general(h, w1_ref[...], (((1,), (1,)), ((), ())),
                            preferred_element_type=jnp.float32)
        h = jnp.maximum(h + b1_ref[...], 0.0)
        o = lax.dot_general(h, w2_ref[...], (((1,), (1,)), ((), ())),
                            preferred_element_type=jnp.float32)
        o_ref[...] = o + b2_ref[...]

    return pl.pallas_call(
        body,
        grid=(N // BN,),
        in_specs=[
            pl.BlockSpec((BN, D), lambda i: (i, 0)),
            pl.BlockSpec((_NC, BN, D), lambda i: (0, i, 0)),
            pl.BlockSpec((D, D), lambda i: (0, 0)),
            pl.BlockSpec((1, D), lambda i: (0, 0)),
            pl.BlockSpec((D, D), lambda i: (0, 0)),
            pl.BlockSpec((1, D), lambda i: (0, 0)),
        ],
        out_specs=pl.BlockSpec((BN, D), lambda i: (i, 0)),
        out_shape=jax.ShapeDtypeStruct((N, D), jnp.float32),
    )(x, parts, W1, b1.reshape(1, D), W2, b2.reshape(1, D))


def kernel(x, edge_index, edge_attr, W1, b1, W2, b2):
    src = edge_index[0].astype(jnp.int32)
    dst = edge_index[1].astype(jnp.int32)
    parts = _sc_aggregate(x, src, dst, edge_attr)
    return _mlp(x, parts, W1, b1, W2, b2)


# speedup vs baseline: 1.0312x; 1.0312x over previous
import functools

import jax
import jax.numpy as jnp
from jax import lax
from jax.experimental import pallas as pl
from jax.experimental.pallas import tpu as pltpu
from jax.experimental.pallas import tpu_sc as plsc

_NC = 2
_NS = 16
_K = 128


def _sc_aggregate(x, srcA, srcB, dst, edge_attr):
    N, D = x.shape
    E = edge_attr.shape[0]
    K = _K
    W = _NC * _NS
    NCH = E // K
    JMAX = (NCH + W - 1) // W
    VEC = D // 16
    ZK = 80
    ZB = N // ZK
    ZPT = (ZB + _NS - 1) // _NS

    mesh = plsc.VectorSubcoreMesh(core_axis_name="c", subcore_axis_name="s",
                                  num_cores=_NC, num_subcores=_NS)

    @functools.partial(
        pl.kernel,
        out_type=jax.ShapeDtypeStruct((_NC, N, D), jnp.float32),
        mesh=mesh,
        scratch_types=[
            pltpu.VMEM((K // 2,), jnp.int32),
            pltpu.VMEM((K // 2,), jnp.int32),
            pltpu.VMEM((K,), jnp.int32),
            pltpu.VMEM((K, D), jnp.float32),
            pltpu.VMEM((K, D), jnp.float32),
            pltpu.VMEM((K, D), jnp.float32),
            pltpu.SemaphoreType.DMA,
            pltpu.SemaphoreType.DMA,
            pltpu.SemaphoreType.DMA,
            pltpu.SemaphoreType.DMA,
            pltpu.SemaphoreType.DMA,
            pltpu.SemaphoreType.DMA,
            pltpu.SemaphoreType.DMA,
            pltpu.SemaphoreType.DMA,
            pltpu.VMEM_SHARED((N, D), jnp.float32),
        ],
    )
    def agg(x_hbm, srcA_hbm, srcB_hbm, dst_hbm, ea_hbm, parts_hbm,
            srcA_k, srcB_k, dst_k, ea0, ea1, xg_v,
            siA, siB, sd, se0, se1, sgA, sgB, ss, acc):
        c = lax.axis_index("c")
        s = lax.axis_index("s")
        w = c * _NS + s

        ea_v = (ea0, ea1)
        se = (se0, se1)
        xga = xg_v.at[pl.ds(0, K // 2)]
        xgb = xg_v.at[pl.ds(K // 2, K // 2)]

        zero = jnp.zeros((16,), jnp.float32)

        def zrow(r, carry):
            for t in range(VEC):
                xg_v[r, pl.ds(t * 16, 16)] = zero
            return carry

        lax.fori_loop(0, ZK, zrow, 0)
        for i in range(ZPT):
            blk = s * ZPT + i
            @pl.when(blk < ZB)
            def _():
                pltpu.sync_copy(xg_v.at[pl.ds(0, ZK)],
                                acc.at[pl.ds(blk * ZK, ZK)])
        plsc.subcore_barrier()

        @pl.when(w < NCH)
        def _():
            pltpu.async_copy(srcA_hbm.at[w, 0], srcA_k, siA)
            pltpu.async_copy(srcB_hbm.at[w, 0], srcB_k, siB)
            pltpu.async_copy(dst_hbm.at[pl.ds(w * K, K)], dst_k, sd)
            pltpu.async_copy(ea_hbm.at[pl.ds(w * K, K)], ea0, se0)

        def chunk_body(j, carry):
            cid = j * W + w
            nbase = (cid + W) * K
            even = lax.rem(j, 2) == 0

            @pl.when((j > 0) & (cid - W < NCH))
            def _():
                pltpu.make_async_copy(xg_v, acc.at[dst_k], ss).wait()
                @pl.when(cid < NCH)
                def _():
                    pltpu.async_copy(dst_hbm.at[pl.ds(cid * K, K)],
                                     dst_k, sd)

            @pl.when(cid < NCH)
            def _():
                pltpu.make_async_copy(srcA_hbm.at[cid, 0],
                                      srcA_k, siA).wait()
                pltpu.make_async_copy(srcB_hbm.at[cid, 0],
                                      srcB_k, siB).wait()
                ga = pltpu.async_copy(x_hbm.at[srcA_k], xga, sgA)
                gb = pltpu.async_copy(x_hbm.at[srcB_k], xgb, sgB)
                ga.wait()
                gb.wait()

                @pl.when(cid + W < NCH)
                def _():
                    pltpu.async_copy(srcA_hbm.at[cid + W, 0], srcA_k, siA)
                    pltpu.async_copy(srcB_hbm.at[cid + W, 0], srcB_k, siB)

                def relu_from(ea_ref):
                    def rrow(r, inner):
                        for t in range(VEC):
                            slc = pl.ds(t * 16, 16)
                            xg_v[r, slc] = jnp.maximum(
                                xg_v[r, slc] + ea_ref[r, slc], 0.0)
                        return inner
                    lax.fori_loop(0, K, rrow, 0)

                @pl.when(even)
                def _():
                    pltpu.make_async_copy(ea_hbm.at[pl.ds(cid * K, K)],
                                          ea0, se0).wait()
                    @pl.when(cid + W < NCH)
                    def _():
                        pltpu.async_copy(ea_hbm.at[pl.ds(nbase, K)], ea1, se1)
                    relu_from(ea0)

                @pl.when(jnp.logical_not(even))
                def _():
                    pltpu.make_async_copy(ea_hbm.at[pl.ds(cid * K, K)],
                                          ea1, se1).wait()
                    @pl.when(cid + W < NCH)
                    def _():
                        pltpu.async_copy(ea_hbm.at[pl.ds(nbase, K)], ea0, se0)
                    relu_from(ea1)

                pltpu.make_async_copy(dst_hbm.at[pl.ds(cid * K, K)],
                                      dst_k, sd).wait()
                pltpu.async_copy(xg_v, acc.at[dst_k], ss, add=True)

            return carry

        lax.fori_loop(0, JMAX + 1, chunk_body, 0)

        plsc.subcore_barrier()
        for i in range(ZPT):
            blk = s * ZPT + i
            @pl.when(blk < ZB)
            def _():
                pltpu.sync_copy(acc.at[pl.ds(blk * ZK, ZK)],
                                parts_hbm.at[c, pl.ds(blk * ZK, ZK)])

    return agg(x, srcA, srcB, dst, edge_attr)


def _mlp(x, parts, W1, b1, W2, b2, BN=1000):
    N, D = x.shape

    def body(x_ref, p_ref, w1_ref, b1_ref, w2_ref, b2_ref, o_ref):
        h = x_ref[...] + p_ref[0] + p_ref[1]
        h = lax.dot_general(h, w1_ref[...], (((1,), (1,)), ((), ())),
                            preferred_element_type=jnp.float32)
        h = jnp.maximum(h + b1_ref[...], 0.0)
        o = lax.dot_general(h, w2_ref[...], (((1,), (1,)), ((), ())),
                            preferred_element_type=jnp.float32)
        o_ref[...] = o + b2_ref[...]

    return pl.pallas_call(
        body,
        grid=(N // BN,),
        in_specs=[
            pl.BlockSpec((BN, D), lambda i: (i, 0)),
            pl.BlockSpec((_NC, BN, D), lambda i: (0, i, 0)),
            pl.BlockSpec((D, D), lambda i: (0, 0)),
            pl.BlockSpec((1, D), lambda i: (0, 0)),
            pl.BlockSpec((D, D), lambda i: (0, 0)),
            pl.BlockSpec((1, D), lambda i: (0, 0)),
        ],
        out_specs=pl.BlockSpec((BN, D), lambda i: (i, 0)),
        out_shape=jax.ShapeDtypeStruct((N, D), jnp.float32),
    )(x, parts, W1, b1.reshape(1, D), W2, b2.reshape(1, D))


def kernel(x, edge_index, edge_attr, W1, b1, W2, b2):
    E = edge_attr.shape[0]
    src2 = edge_index[0].astype(jnp.int32).reshape(E // _K, 2, _K // 2)
    srcA = src2[:, 0:1]
    srcB = src2[:, 1:2]
    dst = edge_index[1].astype(jnp.int32)
    parts = _sc_aggregate(x, srcA, srcB, dst, edge_attr)
    return _mlp(x, parts, W1, b1, W2, b2)
